# baseline (speedup 1.0000x reference)
import jax
import jax.numpy as jnp
from jax.experimental import pallas as pl
from jax.experimental.pallas import tpu as pltpu

DIM = 4096
INTER = 14336
T = 32
S = 4
B1 = 128
N1 = INTER // (S * B1)
B2 = 64
N2 = DIM // (S * B2)


def _ffn_kernel(xt_ref, w1a, w1b, w1c, w1d, w2a, w2b, w2c, w2d, o_ref, h_ref):
    i = pl.program_id(0)
    w1s = (w1a, w1b, w1c, w1d)
    w2s = (w2a, w2b, w2c, w2d)

    @pl.when(i < N1)
    def _phase1():
        for j in range(S):
            h = jax.lax.dot_general(
                w1s[j][...], xt_ref[...],
                dimension_numbers=(((1,), (0,)), ((), ())),
                preferred_element_type=jnp.float32,
            )
            h_ref[pl.ds(i * S * B1 + j * B1, B1), :] = jnp.maximum(h, 0.0)

    @pl.when(i >= N1)
    def _phase2():
        for j in range(S):
            o_ref[j * B2:(j + 1) * B2, :] = jax.lax.dot_general(
                w2s[j][...], h_ref[...],
                dimension_numbers=(((1,), (0,)), ((), ())),
                preferred_element_type=jnp.float32,
            )


@jax.jit
def kernel(x, W1, W2):
    b, t, d = x.shape
    xt = x.reshape(b * t, d).T

    def w1spec(j):
        return pl.BlockSpec(
            (B1, DIM), lambda i, j=j: (jnp.minimum(i, N1 - 1) * S + j, 0))

    def w2spec(j):
        return pl.BlockSpec(
            (B2, INTER), lambda i, j=j: (jnp.maximum(i - N1, 0) * S + j, 0))

    out_t = pl.pallas_call(
        _ffn_kernel,
        grid=(N1 + N2,),
        in_specs=[pl.BlockSpec((DIM, T), lambda i: (0, 0))]
        + [w1spec(j) for j in range(S)]
        + [w2spec(j) for j in range(S)],
        out_specs=pl.BlockSpec((S * B2, T), lambda i: (jnp.maximum(i - N1, 0), 0)),
        out_shape=jax.ShapeDtypeStruct((DIM, T), jnp.float32),
        compiler_params=pltpu.CompilerParams(vmem_limit_bytes=64 * 1024 * 1024),
        scratch_shapes=[pltpu.MemorySpace.VMEM((INTER, T), jnp.float32)],
    )(xt, W1, W1, W1, W1, W2, W2, W2, W2)
    return out_t.T.reshape(b, t, d)


# speedup vs baseline: 1.0211x; 1.0211x over previous
import jax
import jax.numpy as jnp
from jax.experimental import pallas as pl

DIM = 4096
INTER = 14336
BLK = 512


def _ffn_kernel(x_ref, w1_ref, w2_ref, o_ref):
    @pl.when(pl.program_id(0) == 0)
    def _init():
        o_ref[...] = jnp.zeros_like(o_ref)

    h = jax.lax.dot_general(
        x_ref[...], w1_ref[...],
        dimension_numbers=(((1,), (1,)), ((), ())),
        preferred_element_type=jnp.float32,
    )
    h = jnp.maximum(h, 0.0)
    o_ref[...] += jax.lax.dot_general(
        h, w2_ref[...],
        dimension_numbers=(((1,), (1,)), ((), ())),
        preferred_element_type=jnp.float32,
    )


@jax.jit
def kernel(x, W1, W2):
    b, t, d = x.shape
    xt = x.reshape(b * t, d)
    out = pl.pallas_call(
        _ffn_kernel,
        grid=(INTER // BLK,),
        in_specs=[
            pl.BlockSpec((b * t, DIM), lambda i: (0, 0)),
            pl.BlockSpec((BLK, DIM), lambda i: (i, 0)),
            pl.BlockSpec((DIM, BLK), lambda i: (0, i)),
        ],
        out_specs=pl.BlockSpec((b * t, DIM), lambda i: (0, 0)),
        out_shape=jax.ShapeDtypeStruct((b * t, DIM), jnp.float32),
    )(xt, W1, W2)
    return out.reshape(b, t, d)
